# baseline (speedup 1.0000x reference)
import functools

import jax
import jax.numpy as jnp
from jax import lax
from jax.experimental import pallas as pl
from jax.experimental.pallas import tpu as pltpu
from jax.experimental.pallas import tpu_sc as plsc

N_ATOMS = 1000
NC, NS = 2, 16
NW = NC * NS
APW = 32
LAST_BASE = N_ATOMS - APW
L = 16
M = 32
CH = 32
NB = 16
MOMW = 24
CUTOFF = 5.0

_HALF_PI_OVER_CUT = 3.14159265358979 / (2.0 * CUTOFF)


def _rsqrt16(x):
    i = lax.bitcast_convert_type(x, jnp.int32)
    i = jnp.int32(0x5F3759DF) - lax.shift_right_arithmetic(i, 1)
    y = lax.bitcast_convert_type(i, jnp.float32)
    for _ in range(2):
        y = y * (1.5 - 0.5 * x * y * y)
    return y


def _cos16(u):
    u2 = u * u
    return 1.0 + u2 * (-0.5 + u2 * (1.0 / 24.0 + u2 * (-1.0 / 720.0
           + u2 * (1.0 / 40320.0 - u2 * (1.0 / 3628800.0)))))


def _sc_body(coord_h, nbr_h, wmu_h, in0_h, in1_h, in2_h,
             out0_h, out1_h, out2_h,
             coord_v, nbr_v, wmu_v, in0_v, in1_v, in2_v,
             out0_v, out1_v, out2_v, mom_v):
    wid = lax.axis_index("s") * NC + lax.axis_index("c")
    base = jnp.minimum(wid * APW, LAST_BASE)
    pltpu.sync_copy(coord_h, coord_v)
    pltpu.sync_copy(nbr_h.at[wid], nbr_v)
    pltpu.sync_copy(wmu_h, wmu_v)
    pltpu.sync_copy(in0_h.at[wid], in0_v)
    pltpu.sync_copy(in1_h.at[wid], in1_v)
    pltpu.sync_copy(in2_h.at[wid], in2_v)

    iot = lax.iota(jnp.int32, L)
    for g in range(APW // L):
        lb = g * L
        gbase = base + lb
        cx = coord_v[pl.ds(gbase, L)]
        cy = coord_v[pl.ds(N_ATOMS + gbase, L)]
        cz = coord_v[pl.ds(2 * N_ATOMS + gbase, L)]
        bn = (iot + lb) * M

        def m_body(m, acc):
            f0, f1x, f1y, f1z, fxx, fxy, fxz, fyy, fyz, fzz = acc
            idx = plsc.load_gather(nbr_v, [bn + m])
            gx = plsc.load_gather(coord_v, [idx])
            gy = plsc.load_gather(coord_v, [idx + N_ATOMS])
            gz = plsc.load_gather(coord_v, [idx + 2 * N_ATOMS])
            rx = gx - cx
            ry = gy - cy
            rz = gz - cz
            d2 = rx * rx + ry * ry + rz * rz + 1e-10
            rinv = _rsqrt16(d2)
            d = d2 * rinv
            cu = _cos16(jnp.minimum(d, CUTOFF) * _HALF_PI_OVER_CUT)
            fc = cu * cu
            bsum = jnp.zeros((L,), jnp.float32)
            for b in range(NB):
                t = d - wmu_v[0, b, :]
                bsum = bsum + wmu_v[1, b, :] * jnp.exp(-(t * t))
            fn = bsum * fc
            fnx = fn * rx
            fny = fn * ry
            fnz = fn * rz
            return (f0 + fn, f1x + fnx, f1y + fny, f1z + fnz,
                    fxx + fnx * rx, fxy + fnx * ry, fxz + fnx * rz,
                    fyy + fny * ry, fyz + fny * rz, fzz + fnz * rz)

        z = jnp.zeros((L,), jnp.float32)
        F = lax.fori_loop(0, M, m_body, (z,) * 10)
        brow = (iot + lb) * MOMW
        for j in range(10):
            plsc.store_scatter(mom_v, [brow + j], F[j])

    iot3 = iot * 3
    iot9 = iot * 9

    def a_body(a, _):
        fv = mom_v[pl.ds(a * MOMW, L)]
        f = [fv[j] for j in range(10)]
        F0 = f[0]
        F1 = (f[1], f[2], f[3])
        F2 = ((f[4], f[5], f[6]), (f[5], f[7], f[8]), (f[6], f[8], f[9]))
        av = jnp.full((L,), a, jnp.int32)
        for cg in range(CH // L):
            chb = cg * L
            chv = iot + chb
            a0 = in0_v[pl.ds(a * CH + chb, L)]
            a1 = [plsc.load_gather(in1_v, [iot3 + (a * (CH * 3) + chb * 3 + p)])
                  for p in range(3)]
            a2 = [[plsc.load_gather(
                       in2_v, [iot9 + (a * (CH * 9) + chb * 9 + 3 * p + q)])
                   for q in range(3)] for p in range(3)]
            o0 = a0 * F0
            for p in range(3):
                o0 = o0 + a1[p] * F1[p]
                for q in range(3):
                    o0 = o0 + a2[p][q] * F2[p][q]
            out0_v[a, pl.ds(chb, L)] = o0
            for p in range(3):
                o1 = a0 * F1[p] + a1[p] * F0
                for k in range(3):
                    o1 = o1 + a1[k] * F2[k][p] + a2[p][k] * F1[k]
                plsc.store_scatter(
                    out1_v, [av, chv, jnp.full((L,), p, jnp.int32)], o1)
            for p in range(3):
                for q in range(3):
                    o2 = a0 * F2[p][q] + a1[p] * F1[q] + a2[p][q] * F0
                    for k in range(3):
                        o2 = o2 + a2[p][k] * F2[k][q]
                    plsc.store_scatter(
                        out2_v,
                        [av, chv, jnp.full((L,), p, jnp.int32),
                         jnp.full((L,), q, jnp.int32)], o2)
        return 0

    lax.fori_loop(0, APW, a_body, 0)

    pltpu.sync_copy(out0_v, out0_h.at[wid])
    pltpu.sync_copy(out1_v, out1_h.at[wid])
    pltpu.sync_copy(out2_v, out2_h.at[wid])


@functools.partial(
    pl.kernel,
    out_type=(
        jax.ShapeDtypeStruct((NW, APW, CH), jnp.float32),
        jax.ShapeDtypeStruct((NW, APW, CH, 3), jnp.float32),
        jax.ShapeDtypeStruct((NW, APW, CH, 3, 3), jnp.float32),
    ),
    mesh=plsc.VectorSubcoreMesh(core_axis_name="c", subcore_axis_name="s"),
    compiler_params=pltpu.CompilerParams(needs_layout_passes=False,
                                         use_tc_tiling_on_sc=False),
    scratch_types=[
        pltpu.VMEM((3 * N_ATOMS,), jnp.float32),
        pltpu.VMEM((APW * M,), jnp.int32),
        pltpu.VMEM((2, NB, L), jnp.float32),
        pltpu.VMEM((APW * CH,), jnp.float32),
        pltpu.VMEM((APW * CH * 3,), jnp.float32),
        pltpu.VMEM((APW * CH * 9,), jnp.float32),
        pltpu.VMEM((APW, CH), jnp.float32),
        pltpu.VMEM((APW, CH, 3), jnp.float32),
        pltpu.VMEM((APW, CH, 3, 3), jnp.float32),
        pltpu.VMEM((APW * MOMW,), jnp.float32),
    ],
)
def _sc_kernel(coord_h, nbr_h, wmu_h, in0_h, in1_h, in2_h,
               out0_h, out1_h, out2_h,
               coord_v, nbr_v, wmu_v, in0_v, in1_v, in2_v,
               out0_v, out1_v, out2_v, mom_v):
    _sc_body(coord_h, nbr_h, wmu_h, in0_h, in1_h, in2_h,
             out0_h, out1_h, out2_h,
             coord_v, nbr_v, wmu_v, in0_v, in1_v, in2_v,
             out0_v, out1_v, out2_v, mom_v)


def _chunk(x2d, rec):
    return jnp.concatenate(
        [x2d[:LAST_BASE + APW - 8], x2d[LAST_BASE:]]).reshape(NW, APW * rec)


def kernel(input_tensors_0, input_tensors_1, input_tensors_2,
           coordinate, neighbor, mask, rbf_w, rbf_mu):
    coord_t = coordinate[0].T.reshape(3 * N_ATOMS)
    nbr_c = _chunk(neighbor[0], M)
    in0_c = _chunk(input_tensors_0[0], CH)
    in1_c = _chunk(input_tensors_1[0].reshape(N_ATOMS, CH * 3), CH * 3)
    in2_c = _chunk(input_tensors_2[0].reshape(N_ATOMS, CH * 9), CH * 9)
    wmu = jnp.stack([
        jnp.tile(rbf_mu[:, None], (1, L)),
        jnp.tile(rbf_w[:, None], (1, L)),
    ]).astype(jnp.float32)

    out0, out1, out2 = _sc_kernel(coord_t, nbr_c, wmu, in0_c, in1_c, in2_c)
    return (out0, out1, out2)


# speedup vs baseline: 2.9282x; 2.9282x over previous
import functools

import jax
import jax.numpy as jnp
from jax import lax
from jax.experimental import pallas as pl
from jax.experimental.pallas import tpu as pltpu
from jax.experimental.pallas import tpu_sc as plsc

N_ATOMS = 1000
NA = 1024
NC, NS = 2, 16
NW = NC * NS
APW = NA // NW
L = 16
M = 32
CH = 32
NB = 16
CUTOFF = 5.0

_HALF_PI_OVER_CUT = 3.14159265358979 / (2.0 * CUTOFF)


def _rsqrt16(x):
    i = lax.bitcast_convert_type(x, jnp.int32)
    i = jnp.int32(0x5F3759DF) - lax.shift_right_arithmetic(i, 1)
    y = lax.bitcast_convert_type(i, jnp.float32)
    for _ in range(2):
        y = y * (1.5 - 0.5 * x * y * y)
    return y


def _cos16(u):
    u2 = u * u
    return 1.0 + u2 * (-0.5 + u2 * (1.0 / 24.0 + u2 * (-1.0 / 720.0
           + u2 * (1.0 / 40320.0 - u2 * (1.0 / 3628800.0)))))


def _sc_body(coord_h, nbr_h, wmu_h, in0_h, in1_h, in2_h,
             out0_h, out1_h, out2_h,
             coord_v, nbr_v, wmu_v, in0_v, in1_v, in2_v,
             out0_v, out1_v, out2_v):
    wid = lax.axis_index("s") * NC + lax.axis_index("c")
    pltpu.sync_copy(coord_h, coord_v)
    pltpu.sync_copy(nbr_h.at[wid], nbr_v)
    pltpu.sync_copy(wmu_h, wmu_v)
    pltpu.sync_copy(in0_h.at[wid], in0_v)
    pltpu.sync_copy(in1_h.at[wid], in1_v)
    pltpu.sync_copy(in2_h.at[wid], in2_v)

    for g in range(APW // L):
        lb = g * L
        gbase = wid * APW + lb
        cx = coord_v[pl.ds(gbase, L)]
        cy = coord_v[pl.ds(NA + gbase, L)]
        cz = coord_v[pl.ds(2 * NA + gbase, L)]

        def m_body(m, acc):
            f0, f1x, f1y, f1z, fxx, fxy, fxz, fyy, fyz, fzz = acc
            idx = nbr_v[m, pl.ds(lb, L)]
            gx = plsc.load_gather(coord_v, [idx])
            gy = plsc.load_gather(coord_v, [idx + NA])
            gz = plsc.load_gather(coord_v, [idx + 2 * NA])
            rx = gx - cx
            ry = gy - cy
            rz = gz - cz
            d2 = rx * rx + ry * ry + rz * rz + 1e-10
            rinv = _rsqrt16(d2)
            d = d2 * rinv
            cu = _cos16(jnp.minimum(d, CUTOFF) * _HALF_PI_OVER_CUT)
            fc = cu * cu
            bsum = jnp.zeros((L,), jnp.float32)
            for b in range(NB):
                t = d - wmu_v[0, b, :]
                bsum = bsum + wmu_v[1, b, :] * jnp.exp(-(t * t))
            fn = bsum * fc
            fnx = fn * rx
            fny = fn * ry
            fnz = fn * rz
            return (f0 + fn, f1x + fnx, f1y + fny, f1z + fnz,
                    fxx + fnx * rx, fxy + fnx * ry, fxz + fnx * rz,
                    fyy + fny * ry, fyz + fny * rz, fzz + fnz * rz)

        z = jnp.zeros((L,), jnp.float32)
        F0, F1x, F1y, F1z, Fxx, Fxy, Fxz, Fyy, Fyz, Fzz = lax.fori_loop(
            0, M, m_body, (z, z, z, z, z, z, z, z, z, z))
        F1 = (F1x, F1y, F1z)
        F2 = ((Fxx, Fxy, Fxz), (Fxy, Fyy, Fyz), (Fxz, Fyz, Fzz))

        def ch_body(ch, _):
            a0 = in0_v[ch, pl.ds(lb, L)]
            a1 = [in1_v[p, ch, pl.ds(lb, L)] for p in range(3)]
            a2 = [[in2_v[3 * p + q, ch, pl.ds(lb, L)] for q in range(3)]
                  for p in range(3)]
            o0 = a0 * F0
            for p in range(3):
                o0 = o0 + a1[p] * F1[p]
                for q in range(3):
                    o0 = o0 + a2[p][q] * F2[p][q]
            out0_v[ch, pl.ds(lb, L)] = o0
            for p in range(3):
                o1 = a0 * F1[p] + a1[p] * F0
                for k in range(3):
                    o1 = o1 + a1[k] * F2[k][p] + a2[p][k] * F1[k]
                out1_v[p, ch, pl.ds(lb, L)] = o1
            for p in range(3):
                for q in range(3):
                    o2 = a0 * F2[p][q] + a1[p] * F1[q] + a2[p][q] * F0
                    for k in range(3):
                        o2 = o2 + a2[p][k] * F2[k][q]
                    out2_v[3 * p + q, ch, pl.ds(lb, L)] = o2
            return 0

        lax.fori_loop(0, CH, ch_body, 0)

    pltpu.sync_copy(out0_v, out0_h.at[wid])
    pltpu.sync_copy(out1_v, out1_h.at[wid])
    pltpu.sync_copy(out2_v, out2_h.at[wid])


@functools.partial(
    pl.kernel,
    out_type=(
        jax.ShapeDtypeStruct((NW, CH, APW), jnp.float32),
        jax.ShapeDtypeStruct((NW, 3, CH, APW), jnp.float32),
        jax.ShapeDtypeStruct((NW, 9, CH, APW), jnp.float32),
    ),
    mesh=plsc.VectorSubcoreMesh(core_axis_name="c", subcore_axis_name="s"),
    compiler_params=pltpu.CompilerParams(needs_layout_passes=False),
    scratch_types=[
        pltpu.VMEM((3 * NA,), jnp.float32),
        pltpu.VMEM((M, APW), jnp.int32),
        pltpu.VMEM((2, NB, L), jnp.float32),
        pltpu.VMEM((CH, APW), jnp.float32),
        pltpu.VMEM((3, CH, APW), jnp.float32),
        pltpu.VMEM((9, CH, APW), jnp.float32),
        pltpu.VMEM((CH, APW), jnp.float32),
        pltpu.VMEM((3, CH, APW), jnp.float32),
        pltpu.VMEM((9, CH, APW), jnp.float32),
    ],
)
def _sc_kernel(coord_h, nbr_h, wmu_h, in0_h, in1_h, in2_h,
               out0_h, out1_h, out2_h,
               coord_v, nbr_v, wmu_v, in0_v, in1_v, in2_v,
               out0_v, out1_v, out2_v):
    _sc_body(coord_h, nbr_h, wmu_h, in0_h, in1_h, in2_h,
             out0_h, out1_h, out2_h,
             coord_v, nbr_v, wmu_v, in0_v, in1_v, in2_v,
             out0_v, out1_v, out2_v)


def kernel(input_tensors_0, input_tensors_1, input_tensors_2,
           coordinate, neighbor, mask, rbf_w, rbf_mu):
    pad = NA - N_ATOMS
    coord = jnp.pad(coordinate[0], ((0, pad), (0, 0)))
    coord_t = coord.T.reshape(3 * NA)
    nbr = jnp.pad(neighbor[0], ((0, pad), (0, 0)))
    nbr_c = nbr.reshape(NW, APW, M).transpose(0, 2, 1)
    in0 = jnp.pad(input_tensors_0[0], ((0, pad), (0, 0)))
    in0_c = in0.reshape(NW, APW, CH).transpose(0, 2, 1)
    in1 = jnp.pad(input_tensors_1[0], ((0, pad), (0, 0), (0, 0)))
    in1_c = in1.reshape(NW, APW, CH, 3).transpose(0, 3, 2, 1)
    in2 = jnp.pad(input_tensors_2[0].reshape(N_ATOMS, CH, 9),
                  ((0, pad), (0, 0), (0, 0)))
    in2_c = in2.reshape(NW, APW, CH, 9).transpose(0, 3, 2, 1)
    wmu = jnp.stack([
        jnp.tile(rbf_mu[:, None], (1, L)),
        jnp.tile(rbf_w[:, None], (1, L)),
    ]).astype(jnp.float32)

    out0_c, out1_c, out2_c = _sc_kernel(coord_t, nbr_c, wmu,
                                        in0_c, in1_c, in2_c)

    out0 = out0_c.transpose(0, 2, 1).reshape(NA, CH)[:N_ATOMS][None]
    out1 = out1_c.transpose(0, 3, 2, 1).reshape(NA, CH, 3)[:N_ATOMS][None]
    out2 = (out2_c.transpose(0, 3, 2, 1).reshape(NA, CH, 9)[:N_ATOMS]
            .reshape(N_ATOMS, CH, 3, 3)[None])
    return (out0, out1, out2)


# speedup vs baseline: 3.7093x; 1.2668x over previous
import functools

import jax
import jax.numpy as jnp
from jax import lax
from jax.experimental import pallas as pl
from jax.experimental.pallas import tpu as pltpu
from jax.experimental.pallas import tpu_sc as plsc

N_ATOMS = 1000
NA = 1024
NC, NS = 2, 16
NW = NC * NS
APW = NA // NW
L = 16
M = 32
CH = 32
NB = 16
CUTOFF = 5.0

_HALF_PI_OVER_CUT = 3.14159265358979 / (2.0 * CUTOFF)


def _rsqrt16(x):
    i = lax.bitcast_convert_type(x, jnp.int32)
    i = jnp.int32(0x5F3759DF) - lax.shift_right_arithmetic(i, 1)
    y = lax.bitcast_convert_type(i, jnp.float32)
    for _ in range(2):
        y = y * (1.5 - 0.5 * x * y * y)
    return y


def _cos16(u):
    u2 = u * u
    return 1.0 + u2 * (-0.5 + u2 * (1.0 / 24.0 + u2 * (-1.0 / 720.0
           + u2 * (1.0 / 40320.0 - u2 * (1.0 / 3628800.0)))))


def _sc_body(coord_h, nbr_h, wmu_h, in0_h, in1_h, in2_h,
             out0_h, out1_h, out2_h,
             coord_v, nbr_v, wmu_v, in0_v, in1_v, in2_v,
             out0_v, out1_v, out2_v):
    wid = lax.axis_index("s") * NC + lax.axis_index("c")
    pltpu.sync_copy(coord_h, coord_v)
    pltpu.sync_copy(nbr_h.at[wid], nbr_v)
    pltpu.sync_copy(wmu_h, wmu_v)
    pltpu.sync_copy(in0_h.at[wid], in0_v)
    pltpu.sync_copy(in1_h.at[wid], in1_v)
    pltpu.sync_copy(in2_h.at[wid], in2_v)

    for g in range(APW // L):
        lb = g * L
        gbase = wid * APW + lb
        cx = coord_v[pl.ds(gbase, L)]
        cy = coord_v[pl.ds(NA + gbase, L)]
        cz = coord_v[pl.ds(2 * NA + gbase, L)]

        def m_body(m, acc):
            f0, f1x, f1y, f1z, fxx, fxy, fxz, fyy, fyz, fzz = acc
            idx = nbr_v[m, pl.ds(lb, L)]
            gx = plsc.load_gather(coord_v, [idx])
            gy = plsc.load_gather(coord_v, [idx + NA])
            gz = plsc.load_gather(coord_v, [idx + 2 * NA])
            rx = gx - cx
            ry = gy - cy
            rz = gz - cz
            d2 = rx * rx + ry * ry + rz * rz + 1e-10
            rinv = _rsqrt16(d2)
            d = d2 * rinv
            cu = _cos16(jnp.minimum(d, CUTOFF) * _HALF_PI_OVER_CUT)
            fc = cu * cu
            bsum = jnp.zeros((L,), jnp.float32)
            for b in range(NB):
                t = d - wmu_v[0, b, :]
                bsum = bsum + wmu_v[1, b, :] * jnp.exp(-(t * t))
            fn = bsum * fc
            fnx = fn * rx
            fny = fn * ry
            fnz = fn * rz
            return (f0 + fn, f1x + fnx, f1y + fny, f1z + fnz,
                    fxx + fnx * rx, fxy + fnx * ry, fxz + fnx * rz,
                    fyy + fny * ry, fyz + fny * rz, fzz + fnz * rz)

        z = jnp.zeros((L,), jnp.float32)
        F0, F1x, F1y, F1z, Fxx, Fxy, Fxz, Fyy, Fyz, Fzz = lax.fori_loop(
            0, M, m_body, (z, z, z, z, z, z, z, z, z, z))
        F1 = (F1x, F1y, F1z)
        F2 = ((Fxx, Fxy, Fxz), (Fxy, Fyy, Fyz), (Fxz, Fyz, Fzz))

        def ch_body(ch, _):
            a0 = in0_v[ch, pl.ds(lb, L)]
            a1 = [in1_v[p, ch, pl.ds(lb, L)] for p in range(3)]
            a2 = [[in2_v[3 * p + q, ch, pl.ds(lb, L)] for q in range(3)]
                  for p in range(3)]
            o0 = a0 * F0
            for p in range(3):
                o0 = o0 + a1[p] * F1[p]
                for q in range(3):
                    o0 = o0 + a2[p][q] * F2[p][q]
            out0_v[ch, pl.ds(lb, L)] = o0
            for p in range(3):
                o1 = a0 * F1[p] + a1[p] * F0
                for k in range(3):
                    o1 = o1 + a1[k] * F2[k][p] + a2[p][k] * F1[k]
                out1_v[p, ch, pl.ds(lb, L)] = o1
            for p in range(3):
                for q in range(3):
                    o2 = a0 * F2[p][q] + a1[p] * F1[q] + a2[p][q] * F0
                    for k in range(3):
                        o2 = o2 + a2[p][k] * F2[k][q]
                    out2_v[3 * p + q, ch, pl.ds(lb, L)] = o2
            return 0

        lax.fori_loop(0, CH, ch_body, 0)

    pltpu.sync_copy(out0_v, out0_h.at[wid])
    pltpu.sync_copy(out1_v, out1_h.at[wid])
    pltpu.sync_copy(out2_v, out2_h.at[wid])


@functools.partial(
    pl.kernel,
    out_type=(
        jax.ShapeDtypeStruct((NW, CH, APW), jnp.float32),
        jax.ShapeDtypeStruct((NW, 3, CH, APW), jnp.float32),
        jax.ShapeDtypeStruct((NW, 9, CH, APW), jnp.float32),
    ),
    mesh=plsc.VectorSubcoreMesh(core_axis_name="c", subcore_axis_name="s"),
    compiler_params=pltpu.CompilerParams(needs_layout_passes=False),
    scratch_types=[
        pltpu.VMEM((3 * NA,), jnp.float32),
        pltpu.VMEM((M, APW), jnp.int32),
        pltpu.VMEM((2, NB, L), jnp.float32),
        pltpu.VMEM((CH, APW), jnp.float32),
        pltpu.VMEM((3, CH, APW), jnp.float32),
        pltpu.VMEM((9, CH, APW), jnp.float32),
        pltpu.VMEM((CH, APW), jnp.float32),
        pltpu.VMEM((3, CH, APW), jnp.float32),
        pltpu.VMEM((9, CH, APW), jnp.float32),
    ],
)
def _sc_kernel(coord_h, nbr_h, wmu_h, in0_h, in1_h, in2_h,
               out0_h, out1_h, out2_h,
               coord_v, nbr_v, wmu_v, in0_v, in1_v, in2_v,
               out0_v, out1_v, out2_v):
    _sc_body(coord_h, nbr_h, wmu_h, in0_h, in1_h, in2_h,
             out0_h, out1_h, out2_h,
             coord_v, nbr_v, wmu_v, in0_v, in1_v, in2_v,
             out0_v, out1_v, out2_v)


def kernel(input_tensors_0, input_tensors_1, input_tensors_2,
           coordinate, neighbor, mask, rbf_w, rbf_mu):
    pad = NA - N_ATOMS
    coord = jnp.pad(coordinate[0], ((0, pad), (0, 0)))
    coord_t = coord.T.reshape(3 * NA)
    nbr = jnp.pad(neighbor[0], ((0, pad), (0, 0)))
    nbr_c = nbr.reshape(NW, APW, M).transpose(0, 2, 1)
    in0 = jnp.pad(input_tensors_0[0], ((0, pad), (0, 0)))
    in0_c = in0.reshape(NW, APW, CH).transpose(0, 2, 1)
    in1 = jnp.pad(input_tensors_1[0], ((0, pad), (0, 0), (0, 0)))
    in1_c = in1.reshape(NW, APW, CH, 3).transpose(0, 3, 2, 1)
    in2 = jnp.pad(input_tensors_2[0].reshape(N_ATOMS, CH, 9),
                  ((0, pad), (0, 0), (0, 0)))
    in2_c = in2.reshape(NW, APW, CH, 9).transpose(0, 3, 2, 1)
    wmu = jnp.stack([
        jnp.tile(rbf_mu[:, None], (1, L)),
        jnp.tile(rbf_w[:, None], (1, L)),
    ]).astype(jnp.float32)

    out0_c, out1_c, out2_c = _sc_kernel(coord_t, nbr_c, wmu,
                                        in0_c, in1_c, in2_c)

    return (out0_c, out1_c, out2_c)
